# baseline (speedup 1.0000x reference)
import jax
import jax.numpy as jnp
from jax.experimental import pallas as pl
from jax.experimental.pallas import tpu as pltpu

_CHUNK_ROWS = 1024
_NBUF = 8
_PREFETCH = 4


def kernel(x, table):
    del x
    max_len, emb_dim = table.shape
    nch = max_len // _CHUNK_ROWS

    def body(in_hbm, out_hbm, buf, *sems):
        sin = sems[:_NBUF]
        sout = sems[_NBUF:]

        def cin(i):
            return pltpu.make_async_copy(
                in_hbm.at[pl.ds(i * _CHUNK_ROWS, _CHUNK_ROWS)],
                buf.at[i % _NBUF],
                sin[i % _NBUF],
            )

        def cout(i):
            return pltpu.make_async_copy(
                buf.at[i % _NBUF],
                out_hbm.at[pl.ds(i * _CHUNK_ROWS, _CHUNK_ROWS)],
                sout[i % _NBUF],
            )

        for i in range(min(_PREFETCH, nch)):
            cin(i).start()
        for i in range(nch):
            cin(i).wait()
            cout(i).start()
            j = i + _PREFETCH
            if j < nch:
                if j >= _NBUF:
                    cout(j - _NBUF).wait()
                cin(j).start()
        for i in range(max(nch - _NBUF, 0), nch):
            cout(i).wait()

    out = pl.pallas_call(
        body,
        in_specs=[pl.BlockSpec(memory_space=pltpu.MemorySpace.HBM)],
        out_specs=pl.BlockSpec(memory_space=pltpu.MemorySpace.HBM),
        out_shape=jax.ShapeDtypeStruct((max_len, emb_dim), table.dtype),
        scratch_shapes=[pltpu.VMEM((_NBUF, _CHUNK_ROWS, emb_dim), table.dtype)]
        + [pltpu.SemaphoreType.DMA] * (2 * _NBUF),
    )(table)
    return out[None]
